# baseline (speedup 1.0000x reference)
import functools

import jax
import jax.numpy as jnp
from jax import lax
from jax.experimental import pallas as pl
from jax.experimental.pallas import tpu as pltpu
from jax.experimental.pallas import tpu_sc as plsc

VOCAB = 30522
HIDDEN = 768
BATCH = 4
SEQ = 2048
EPS = 1e-12
L = 16
HC = HIDDEN // L


def _hsum(v):
    s = v[0]
    for i in range(1, L):
        s = s + v[i]
    return s


def _rsqrt(x):
    i = lax.bitcast_convert_type(x, jnp.int32)
    i = jnp.int32(0x5F3759DF) - lax.shift_right_logical(i, jnp.int32(1))
    y = lax.bitcast_convert_type(i, jnp.float32)
    for _ in range(4):
        y = y * (1.5 - 0.5 * x * y * y)
    return y


def _body(nc, spt, ids_hbm, ttf_hbm, word_hbm, pos_hbm, type_hbm, gamma_hbm,
          beta_hbm, out_hbm, idx_v, ttf_v, rows_v, pos_v, type_v, diff_v,
          gamma_v, beta_v, sem):
    wid = lax.axis_index("s") * nc + lax.axis_index("c")
    s0 = wid * spt

    pltpu.sync_copy(pos_hbm.at[pl.ds(s0, spt)], pos_v)
    pltpu.sync_copy(type_hbm, type_v)
    pltpu.sync_copy(gamma_hbm, gamma_v)
    pltpu.sync_copy(beta_hbm, beta_v)

    def dloop(c, _):
        off = c * L
        diff_v[pl.ds(off, L)] = (type_v[1, pl.ds(off, L)]
                                 - type_v[0, pl.ds(off, L)])
        return 0
    lax.fori_loop(0, HC, dloop, 0)

    def ploop(i, _):
        def pc(c, _):
            off = c * L
            pos_v[i, pl.ds(off, L)] = (pos_v[i, pl.ds(off, L)]
                                       + type_v[0, pl.ds(off, L)])
            return 0
        lax.fori_loop(0, HC, pc, 0)
        return 0
    lax.fori_loop(0, spt, ploop, 0)

    def batch(b, _):
        pltpu.sync_copy(ids_hbm.at[b, pl.ds(s0, spt)], idx_v)
        pltpu.sync_copy(ttf_hbm.at[b, pl.ds(s0, spt)], ttf_v)
        pltpu.async_copy(word_hbm.at[idx_v], rows_v, sem).wait()

        def tok_group(g, _):
            ttv = ttf_v[pl.ds(g * L, L)]
            for j2 in range(L):
                j = g * L + j2
                ttf = ttv[j2]

                def p1(c, carry):
                    vs, vq = carry
                    off = c * L
                    x = (rows_v[j, pl.ds(off, L)] + pos_v[j, pl.ds(off, L)]
                         + ttf * diff_v[pl.ds(off, L)])
                    rows_v[j, pl.ds(off, L)] = x
                    return (vs + x, vq + x * x)

                zero = jnp.zeros((L,), jnp.float32)
                vs, vq = lax.fori_loop(0, HC, p1, (zero, zero))
                s1 = _hsum(vs)
                s2 = _hsum(vq)
                mean = s1 * (1.0 / HIDDEN)
                var = s2 * (1.0 / HIDDEN) - mean * mean
                rstd = _rsqrt(var + EPS)

                def p2(c, _):
                    off = c * L
                    x = rows_v[j, pl.ds(off, L)]
                    rows_v[j, pl.ds(off, L)] = ((x - mean) * rstd
                                                * gamma_v[pl.ds(off, L)]
                                                + beta_v[pl.ds(off, L)])
                    return 0
                lax.fori_loop(0, HC, p2, 0)
            return 0

        lax.fori_loop(0, spt // L, tok_group, 0)
        pltpu.sync_copy(rows_v, out_hbm.at[b, pl.ds(s0, spt)])
        return 0

    lax.fori_loop(0, BATCH, batch, 0)


def kernel(input_ids, token_type_ids, word_emb, pos_emb, type_emb, ln_gamma,
           ln_beta):
    ids = input_ids.astype(jnp.int32)
    ttf = token_type_ids.astype(jnp.float32)

    try:
        info = plsc.get_sparse_core_info()
        nc, ns = info.num_cores, info.num_subcores
    except Exception:
        nc, ns = 2, 16
    nw = nc * ns
    spt = SEQ // nw

    f = pl.kernel(
        functools.partial(_body, nc, spt),
        out_type=jax.ShapeDtypeStruct((BATCH, SEQ, HIDDEN), jnp.float32),
        mesh=plsc.VectorSubcoreMesh(core_axis_name="c", subcore_axis_name="s"),
        scratch_types=[
            pltpu.VMEM((spt,), jnp.int32),
            pltpu.VMEM((spt,), jnp.float32),
            pltpu.VMEM((spt, HIDDEN), jnp.float32),
            pltpu.VMEM((spt, HIDDEN), jnp.float32),
            pltpu.VMEM((2, HIDDEN), jnp.float32),
            pltpu.VMEM((HIDDEN,), jnp.float32),
            pltpu.VMEM((HIDDEN,), jnp.float32),
            pltpu.VMEM((HIDDEN,), jnp.float32),
            pltpu.SemaphoreType.DMA,
        ],
    )
    return f(ids, ttf, word_emb, pos_emb, type_emb, ln_gamma, ln_beta)


# speedup vs baseline: 1.1565x; 1.1565x over previous
import functools

import jax
import jax.numpy as jnp
from jax import lax
from jax.experimental import pallas as pl
from jax.experimental.pallas import tpu as pltpu
from jax.experimental.pallas import tpu_sc as plsc

VOCAB = 30522
HIDDEN = 768
BATCH = 4
SEQ = 2048
EPS = 1e-12
L = 16
HC = HIDDEN // L
CH = 32
UNROLL = 8


def _hsum(v):
    s = v[0]
    for i in range(1, L):
        s = s + v[i]
    return s


def _rsqrt(x):
    i = lax.bitcast_convert_type(x, jnp.int32)
    i = jnp.int32(0x5F3759DF) - lax.shift_right_logical(i, jnp.int32(1))
    y = lax.bitcast_convert_type(i, jnp.float32)
    for _ in range(3):
        y = y * (1.5 - 0.5 * x * y * y)
    return y


def _body(nc, spt, ids_hbm, ttb_hbm, word_hbm, pos_hbm, type_hbm, gamma_hbm,
          beta_hbm, out_hbm, idx_a, idx_b, ttb_a, ttb_b, rows_a, rows_b,
          pos_v, type_v, diff_v, gamma_v, beta_v, sem_ga, sem_gb, sem_oa,
          sem_ob):
    wid = lax.axis_index("s") * nc + lax.axis_index("c")
    s0 = wid * spt

    pltpu.sync_copy(pos_hbm.at[pl.ds(s0, spt)], pos_v)
    pltpu.sync_copy(type_hbm, type_v)
    pltpu.sync_copy(gamma_hbm, gamma_v)
    pltpu.sync_copy(beta_hbm, beta_v)

    def dloop(c, _):
        off = c * L
        diff_v[pl.ds(off, L)] = (type_v[1, pl.ds(off, L)]
                                 - type_v[0, pl.ds(off, L)])
        return 0
    lax.fori_loop(0, HC, dloop, 0)

    def ploop(i, _):
        def pc(c, _):
            off = c * L
            pos_v[i, pl.ds(off, L)] = (pos_v[i, pl.ds(off, L)]
                                       + type_v[0, pl.ds(off, L)])
            return 0
        lax.fori_loop(0, HC, pc, 0)
        return 0
    lax.fori_loop(0, spt, ploop, 0)

    idx = (idx_a, idx_b)
    ttb = (ttb_a, ttb_b)
    rows = (rows_a, rows_b)
    sem_g = (sem_ga, sem_gb)
    sem_o = (sem_oa, sem_ob)
    nchunks = BATCH * (spt // CH)

    def stage(k, cur):
        b, half = k // (spt // CH), k % (spt // CH)
        tok = s0 + half * CH
        pltpu.sync_copy(ids_hbm.at[b, pl.ds(tok, CH)], idx[cur])
        pltpu.sync_copy(ttb_hbm.at[b, pl.ds(tok, CH)], ttb[cur])
        return pltpu.async_copy(word_hbm.at[idx[cur]], rows[cur], sem_g[cur])

    def compute(k, cur):
        rv, tv = rows[cur], ttb[cur]
        poff = (k % (spt // CH)) * CH

        def tok(j, _):
            ttf = tv[j]

            def p1(cc, carry):
                vs, vq = carry
                for u in range(UNROLL):
                    o = pl.ds(cc * (UNROLL * L) + u * L, L)
                    x = rv[j, o] + pos_v[poff + j, o] + ttf * diff_v[o]
                    rv[j, o] = x
                    vs = vs + x
                    vq = vq + x * x
                return (vs, vq)

            zero = jnp.zeros((L,), jnp.float32)
            vs, vq = lax.fori_loop(0, HC // UNROLL, p1, (zero, zero))
            mean = _hsum(vs) * (1.0 / HIDDEN)
            var = _hsum(vq) * (1.0 / HIDDEN) - mean * mean
            rstd = _rsqrt(var + EPS)

            def p2(cc, _):
                for u in range(UNROLL):
                    o = pl.ds(cc * (UNROLL * L) + u * L, L)
                    x = rv[j, o]
                    rv[j, o] = ((x - mean) * rstd * gamma_v[o] + beta_v[o])
                return 0
            lax.fori_loop(0, HC // UNROLL, p2, 0)
            return 0

        lax.fori_loop(0, CH, tok, 0)

    def writeback(k, cur):
        b, half = k // (spt // CH), k % (spt // CH)
        tok = s0 + half * CH
        return pltpu.async_copy(rows[cur], out_hbm.at[b, pl.ds(tok, CH)],
                                sem_o[cur])

    gh = [None, None]
    oh = [None, None]
    gh[0] = stage(0, 0)
    for k in range(nchunks):
        cur = k % 2
        nxt = 1 - cur
        gh[cur].wait()
        if k + 1 < nchunks:
            if oh[nxt] is not None:
                oh[nxt].wait()
            gh[nxt] = stage(k + 1, nxt)
        compute(k, cur)
        oh[cur] = writeback(k, cur)
    oh[0].wait()
    oh[1].wait()


def kernel(input_ids, token_type_ids, word_emb, pos_emb, type_emb, ln_gamma,
           ln_beta):
    ids = input_ids.astype(jnp.int32)
    ttb = jnp.broadcast_to(token_type_ids.astype(jnp.float32)[..., None],
                           (BATCH, SEQ, L))

    try:
        info = plsc.get_sparse_core_info()
        nc, ns = info.num_cores, info.num_subcores
    except Exception:
        nc, ns = 2, 16
    nw = nc * ns
    spt = SEQ // nw

    f = pl.kernel(
        functools.partial(_body, nc, spt),
        out_type=jax.ShapeDtypeStruct((BATCH, SEQ, HIDDEN), jnp.float32),
        mesh=plsc.VectorSubcoreMesh(core_axis_name="c", subcore_axis_name="s"),
        scratch_types=[
            pltpu.VMEM((CH,), jnp.int32),
            pltpu.VMEM((CH,), jnp.int32),
            pltpu.VMEM((CH, L), jnp.float32),
            pltpu.VMEM((CH, L), jnp.float32),
            pltpu.VMEM((CH, HIDDEN), jnp.float32),
            pltpu.VMEM((CH, HIDDEN), jnp.float32),
            pltpu.VMEM((spt, HIDDEN), jnp.float32),
            pltpu.VMEM((2, HIDDEN), jnp.float32),
            pltpu.VMEM((HIDDEN,), jnp.float32),
            pltpu.VMEM((HIDDEN,), jnp.float32),
            pltpu.VMEM((HIDDEN,), jnp.float32),
            pltpu.SemaphoreType.DMA,
            pltpu.SemaphoreType.DMA,
            pltpu.SemaphoreType.DMA,
            pltpu.SemaphoreType.DMA,
        ],
    )
    return f(ids, ttb, word_emb, pos_emb, type_emb, ln_gamma, ln_beta)
